# baseline (speedup 1.0000x reference)
import functools

import jax
import jax.numpy as jnp
from jax import lax
from jax.experimental import pallas as pl
from jax.experimental.pallas import tpu as pltpu
from jax.experimental.pallas import tpu_sc as plsc

N = 10000
NP = 10240
E = 320000
D_IN = 128
DH = 256
T = 50
H = 4
HD = 64
FF = 512
BQ = 256
NB = NP // BQ
NEG = -1e30

SC_C = 2
SC_S = 16
NW = SC_C * SC_S


def _sc_mesh():
    return plsc.VectorSubcoreMesh(core_axis_name="c", subcore_axis_name="s",
                                  num_cores=SC_C, num_subcores=SC_S)


def _mm_kernel(x_ref, w_ref, b_ref, o_ref, *, act):
    z = jnp.dot(x_ref[...], w_ref[...], preferred_element_type=jnp.float32) + b_ref[...]
    if act == "relu":
        z = jnp.maximum(z, 0.0)
    o_ref[...] = z


def _dense(x, w, b, act=None):
    n, k = x.shape
    m = w.shape[1]
    return pl.pallas_call(
        functools.partial(_mm_kernel, act=act),
        grid=(n // BQ,),
        in_specs=[pl.BlockSpec((BQ, k), lambda i: (i, 0)),
                  pl.BlockSpec((k, m), lambda i: (0, 0)),
                  pl.BlockSpec((1, m), lambda i: (0, 0))],
        out_specs=pl.BlockSpec((BQ, m), lambda i: (i, 0)),
        out_shape=jax.ShapeDtypeStruct((n, m), jnp.float32),
    )(x, w, b.reshape(1, m))


def _stats_block(i, z):
    rows = i * BQ + lax.broadcasted_iota(jnp.int32, (BQ, 1), 0)
    zm = jnp.where(rows < N, z, 0.0)
    return jnp.concatenate(
        [jnp.sum(zm, axis=0, keepdims=True),
         jnp.sum(zm * zm, axis=0, keepdims=True),
         jnp.zeros((6, DH), jnp.float32)], axis=0)


def _sage_mean_kernel(suma_ref, sumb_ref, cnt_ref, h_ref, wl_ref, wr_ref, bl_ref,
                      z_ref, st_ref):
    i = pl.program_id(0)
    sa = suma_ref[0] + suma_ref[1]
    sb = sumb_ref[0] + sumb_ref[1]
    c = cnt_ref[0] + cnt_ref[1]
    cc = jnp.maximum(c[:, 0:1], 1.0)
    agg = jnp.concatenate([sa, sb], axis=1) / cc
    z = (jnp.dot(agg, wl_ref[...], preferred_element_type=jnp.float32)
         + jnp.dot(h_ref[...], wr_ref[...], preferred_element_type=jnp.float32)
         + bl_ref[...])
    z_ref[...] = z
    st = _stats_block(i, z)

    @pl.when(i == 0)
    def _():
        st_ref[...] = st

    @pl.when(i > 0)
    def _():
        st_ref[...] = st_ref[...] + st


def _sage_mean(suma, sumb, cnt, h, wl, wr, bl):
    return pl.pallas_call(
        _sage_mean_kernel,
        grid=(NB,),
        in_specs=[pl.BlockSpec((2, BQ, 128), lambda i: (0, i, 0)),
                  pl.BlockSpec((2, BQ, 128), lambda i: (0, i, 0)),
                  pl.BlockSpec((2, BQ, 128), lambda i: (0, i, 0)),
                  pl.BlockSpec((BQ, DH), lambda i: (i, 0)),
                  pl.BlockSpec((DH, DH), lambda i: (0, 0)),
                  pl.BlockSpec((DH, DH), lambda i: (0, 0)),
                  pl.BlockSpec((1, DH), lambda i: (0, 0))],
        out_specs=[pl.BlockSpec((BQ, DH), lambda i: (i, 0)),
                   pl.BlockSpec((8, DH), lambda i: (0, 0))],
        out_shape=[jax.ShapeDtypeStruct((NP, DH), jnp.float32),
                   jax.ShapeDtypeStruct((8, DH), jnp.float32)],
    )(suma, sumb, cnt, h, wl, wr, bl.reshape(1, DH))


def _sage_max_kernel(aggm_ref, h_ref, wl_ref, wr_ref, bl_ref, z_ref, st_ref):
    i = pl.program_id(0)
    a = aggm_ref[...]
    agg = jnp.where(jnp.isfinite(a), a, 0.0)
    z = (jnp.dot(agg, wl_ref[...], preferred_element_type=jnp.float32)
         + jnp.dot(h_ref[...], wr_ref[...], preferred_element_type=jnp.float32)
         + bl_ref[...])
    z_ref[...] = z
    st = _stats_block(i, z)

    @pl.when(i == 0)
    def _():
        st_ref[...] = st

    @pl.when(i > 0)
    def _():
        st_ref[...] = st_ref[...] + st


def _sage_max(aggm, h, wl, wr, bl):
    return pl.pallas_call(
        _sage_max_kernel,
        grid=(NB,),
        in_specs=[pl.BlockSpec((BQ, DH), lambda i: (i, 0)),
                  pl.BlockSpec((BQ, DH), lambda i: (i, 0)),
                  pl.BlockSpec((DH, DH), lambda i: (0, 0)),
                  pl.BlockSpec((DH, DH), lambda i: (0, 0)),
                  pl.BlockSpec((1, DH), lambda i: (0, 0))],
        out_specs=[pl.BlockSpec((BQ, DH), lambda i: (i, 0)),
                   pl.BlockSpec((8, DH), lambda i: (0, 0))],
        out_shape=[jax.ShapeDtypeStruct((NP, DH), jnp.float32),
                   jax.ShapeDtypeStruct((8, DH), jnp.float32)],
    )(aggm, h, wl, wr, bl.reshape(1, DH))


def _bn_kernel(z_ref, st_ref, g_ref, b_ref, o_ref):
    m = st_ref[0:1, :] * (1.0 / N)
    v = st_ref[1:2, :] * (1.0 / N) - m * m
    o_ref[...] = jnp.maximum(
        g_ref[...] * (z_ref[...] - m) * lax.rsqrt(v + 1e-5) + b_ref[...], 0.0)


def _bn_relu(z, st, g, b):
    return pl.pallas_call(
        _bn_kernel,
        grid=(NB,),
        in_specs=[pl.BlockSpec((BQ, DH), lambda i: (i, 0)),
                  pl.BlockSpec((8, DH), lambda i: (0, 0)),
                  pl.BlockSpec((1, DH), lambda i: (0, 0)),
                  pl.BlockSpec((1, DH), lambda i: (0, 0))],
        out_specs=pl.BlockSpec((BQ, DH), lambda i: (i, 0)),
        out_shape=jax.ShapeDtypeStruct((NP, DH), jnp.float32),
    )(z, st, g.reshape(1, DH), b.reshape(1, DH))


def _bn_temb_kernel(z_ref, st_ref, g_ref, b_ref, t_ref, temb_ref, o_ref):
    m = st_ref[0:1, :] * (1.0 / N)
    v = st_ref[1:2, :] * (1.0 / N) - m * m
    bn = jnp.maximum(
        g_ref[...] * (z_ref[...] - m) * lax.rsqrt(v + 1e-5) + b_ref[...], 0.0)
    oh = (t_ref[...] == lax.broadcasted_iota(jnp.int32, (BQ, 64), 1)).astype(jnp.float32)
    o_ref[...] = bn + jnp.dot(oh, temb_ref[...], preferred_element_type=jnp.float32)


def _bn_relu_temb(z, st, g, b, tcol, temb_pad):
    return pl.pallas_call(
        _bn_temb_kernel,
        grid=(NB,),
        in_specs=[pl.BlockSpec((BQ, DH), lambda i: (i, 0)),
                  pl.BlockSpec((8, DH), lambda i: (0, 0)),
                  pl.BlockSpec((1, DH), lambda i: (0, 0)),
                  pl.BlockSpec((1, DH), lambda i: (0, 0)),
                  pl.BlockSpec((BQ, 1), lambda i: (i, 0)),
                  pl.BlockSpec((64, DH), lambda i: (0, 0))],
        out_specs=pl.BlockSpec((BQ, DH), lambda i: (i, 0)),
        out_shape=jax.ShapeDtypeStruct((NP, DH), jnp.float32),
    )(z, st, g.reshape(1, DH), b.reshape(1, DH), tcol, temb_pad)


def _attn_kernel(klo_ref, knum_ref, q_ref, k_ref, v_ref, segc_ref, segr_ref, o_ref):
    i = pl.program_id(0)
    klo = klo_ref[i]
    knum = knum_ref[i]
    segq = segc_ref[...]
    q = q_ref[...] * jnp.float32(0.125)
    for h in range(H):
        qh = q[:, h * HD:(h + 1) * HD]

        def body(j, carry, _h=h, _qh=qh):
            m, l, acc = carry
            kb = klo + j
            krows = k_ref[pl.ds(kb * BQ, BQ), _h * HD:(_h + 1) * HD]
            s = lax.dot_general(_qh, krows, (((1,), (1,)), ((), ())),
                                preferred_element_type=jnp.float32)
            segk = segr_ref[kb]
            s = jnp.where(segq == segk, s, NEG)
            mnew = jnp.maximum(m, jnp.max(s, axis=1, keepdims=True))
            p = jnp.exp(s - mnew)
            corr = jnp.exp(m - mnew)
            vrows = v_ref[pl.ds(kb * BQ, BQ), _h * HD:(_h + 1) * HD]
            l2 = l * corr + jnp.sum(p, axis=1, keepdims=True)
            acc2 = acc * corr + jnp.dot(p, vrows, preferred_element_type=jnp.float32)
            return mnew, l2, acc2

        m0 = jnp.full((BQ, 1), NEG, jnp.float32)
        l0 = jnp.zeros((BQ, 1), jnp.float32)
        a0 = jnp.zeros((BQ, HD), jnp.float32)
        m, l, acc = lax.fori_loop(0, knum, body, (m0, l0, a0))
        o_ref[:, h * HD:(h + 1) * HD] = acc / l


def _attention(qkv, segc, segr, klo, knum):
    return pl.pallas_call(
        _attn_kernel,
        grid=(NB,),
        in_specs=[pl.BlockSpec(memory_space=pltpu.SMEM),
                  pl.BlockSpec(memory_space=pltpu.SMEM),
                  pl.BlockSpec((BQ, DH), lambda i: (i, 0)),
                  pl.BlockSpec((NP, DH), lambda i: (0, 1)),
                  pl.BlockSpec((NP, DH), lambda i: (0, 2)),
                  pl.BlockSpec((BQ, 1), lambda i: (i, 0)),
                  pl.BlockSpec((NB, 1, BQ), lambda i: (0, 0, 0))],
        out_specs=pl.BlockSpec((BQ, DH), lambda i: (i, 0)),
        out_shape=jax.ShapeDtypeStruct((NP, DH), jnp.float32),
    )(klo, knum, qkv, qkv, qkv, segc, segr)


def _post_kernel(a_ref, x_ref, wo_ref, bo_ref, g1_ref, b1_ref, w1_ref, bb1_ref,
                 w2_ref, bb2_ref, g2_ref, b2_ref, o_ref):
    o = (jnp.dot(a_ref[...], wo_ref[...], preferred_element_type=jnp.float32)
         + bo_ref[...] + x_ref[...])
    mu = jnp.mean(o, axis=1, keepdims=True)
    var = jnp.mean((o - mu) * (o - mu), axis=1, keepdims=True)
    u = g1_ref[...] * (o - mu) * lax.rsqrt(var + 1e-5) + b1_ref[...]
    f = jnp.maximum(
        jnp.dot(u, w1_ref[...], preferred_element_type=jnp.float32) + bb1_ref[...], 0.0)
    f = jnp.dot(f, w2_ref[...], preferred_element_type=jnp.float32) + bb2_ref[...] + u
    mu2 = jnp.mean(f, axis=1, keepdims=True)
    var2 = jnp.mean((f - mu2) * (f - mu2), axis=1, keepdims=True)
    o_ref[...] = g2_ref[...] * (f - mu2) * lax.rsqrt(var2 + 1e-5) + b2_ref[...]


def _post(att, hs, l):
    row = lambda a: a.reshape(1, -1)
    return pl.pallas_call(
        _post_kernel,
        grid=(NB,),
        in_specs=[pl.BlockSpec((BQ, DH), lambda i: (i, 0)),
                  pl.BlockSpec((BQ, DH), lambda i: (i, 0)),
                  pl.BlockSpec((DH, DH), lambda i: (0, 0)),
                  pl.BlockSpec((1, DH), lambda i: (0, 0)),
                  pl.BlockSpec((1, DH), lambda i: (0, 0)),
                  pl.BlockSpec((1, DH), lambda i: (0, 0)),
                  pl.BlockSpec((DH, FF), lambda i: (0, 0)),
                  pl.BlockSpec((1, FF), lambda i: (0, 0)),
                  pl.BlockSpec((FF, DH), lambda i: (0, 0)),
                  pl.BlockSpec((1, DH), lambda i: (0, 0)),
                  pl.BlockSpec((1, DH), lambda i: (0, 0)),
                  pl.BlockSpec((1, DH), lambda i: (0, 0))],
        out_specs=pl.BlockSpec((BQ, DH), lambda i: (i, 0)),
        out_shape=jax.ShapeDtypeStruct((NP, DH), jnp.float32),
    )(att, hs, l['Wo'], row(l['bo']), row(l['ln1_g']), row(l['ln1_b']),
      l['W1'], row(l['b1']), l['W2'], row(l['b2']), row(l['ln2_g']), row(l['ln2_b']))


def _cls_kernel(x_ref, w1_ref, b1_ref, w2_ref, b2_ref, o_ref):
    hh = jnp.maximum(
        jnp.dot(x_ref[...], w1_ref[...], preferred_element_type=jnp.float32)
        + b1_ref[...], 0.0)
    o_ref[...] = jnp.dot(hh, w2_ref[...], preferred_element_type=jnp.float32) + b2_ref[...]


def _cls(x, w1, b1, w2p, b2p):
    return pl.pallas_call(
        _cls_kernel,
        grid=(NB,),
        in_specs=[pl.BlockSpec((BQ, DH), lambda i: (i, 0)),
                  pl.BlockSpec((DH, 64), lambda i: (0, 0)),
                  pl.BlockSpec((1, 64), lambda i: (0, 0)),
                  pl.BlockSpec((64, 128), lambda i: (0, 0)),
                  pl.BlockSpec((1, 128), lambda i: (0, 0))],
        out_specs=pl.BlockSpec((BQ, 128), lambda i: (i, 0)),
        out_shape=jax.ShapeDtypeStruct((NP, 128), jnp.float32),
    )(x, w1, b1.reshape(1, 64), w2p, b2p.reshape(1, 128))


def _sc_segsum(src, dst, ha, hb, zrow, ones128, with_cnt):
    TE = E // NW
    C = 80
    NCH = TE // C
    RZ = NP // SC_S
    outs = [jax.ShapeDtypeStruct((SC_C, NP, 128), jnp.float32),
            jax.ShapeDtypeStruct((SC_C, NP, 128), jnp.float32)]
    if with_cnt:
        outs.append(jax.ShapeDtypeStruct((SC_C, NP, 128), jnp.float32))

    @functools.partial(
        pl.kernel,
        out_type=outs,
        mesh=_sc_mesh(),
        scratch_types=[pltpu.VMEM_SHARED((NP, 128), jnp.float32),
                       pltpu.VMEM((C,), jnp.int32),
                       pltpu.VMEM((C,), jnp.int32),
                       pltpu.VMEM((C, 128), jnp.float32),
                       pltpu.VMEM((C, 128), jnp.float32),
                       pltpu.SemaphoreType.DMA],
    )
    def k(src_h, dst_h, ha_h, hb_h, zrow_h, ones_h, *rest):
        if with_cnt:
            suma_h, sumb_h, cnt_h = rest[:3]
            scr = rest[3:]
        else:
            suma_h, sumb_h = rest[:2]
            scr = rest[2:]
        acc_sp, si_v, di_v, rows_v, ones_v, sem = scr
        c = lax.axis_index("c")
        s = lax.axis_index("s")
        ebase = (c * SC_S + s) * TE
        rz = s * RZ

        def zero_acc():
            pltpu.sync_copy(zrow_h.at[pl.ds(rz, RZ)], acc_sp.at[pl.ds(rz, RZ)])

        def sum_pass(h_h, out_h):
            def body(i, carry):
                off = ebase + i * C
                pltpu.sync_copy(src_h.at[pl.ds(off, C)], si_v)
                pltpu.sync_copy(dst_h.at[pl.ds(off, C)], di_v)
                pltpu.async_copy(h_h.at[si_v], rows_v, sem).wait()
                pltpu.sync_copy(rows_v, acc_sp.at[di_v], add=True)
                return carry

            lax.fori_loop(0, NCH, body, 0)
            plsc.subcore_barrier()
            pltpu.sync_copy(acc_sp.at[pl.ds(rz, RZ)], out_h.at[c, pl.ds(rz, RZ)])
            plsc.subcore_barrier()

        zero_acc()
        plsc.subcore_barrier()
        sum_pass(ha_h, suma_h)
        zero_acc()
        plsc.subcore_barrier()
        sum_pass(hb_h, sumb_h)
        if with_cnt:
            pltpu.sync_copy(ones_h, ones_v)
            zero_acc()
            plsc.subcore_barrier()

            def body_c(i, carry):
                off = ebase + i * C
                pltpu.sync_copy(dst_h.at[pl.ds(off, C)], di_v)
                pltpu.sync_copy(ones_v, acc_sp.at[di_v], add=True)
                return carry

            lax.fori_loop(0, NCH, body_c, 0)
            plsc.subcore_barrier()
            pltpu.sync_copy(acc_sp.at[pl.ds(rz, RZ)], cnt_h.at[c, pl.ds(rz, RZ)])

    return k(src, dst, ha, hb, zrow, ones128)


def _sc_segmax(src, dst, h, ninit):
    RW = NP // NW
    AR = RW + 8
    CH = 4000
    NCH = E // CH
    G = 96

    VPC = CH // 16

    @functools.partial(
        pl.kernel,
        out_type=jax.ShapeDtypeStruct((NP, DH), jnp.float32),
        mesh=_sc_mesh(),
        scratch_types=[pltpu.VMEM((AR, DH), jnp.float32),
                       pltpu.VMEM((CH,), jnp.int32),
                       pltpu.VMEM((CH,), jnp.int32),
                       pltpu.VMEM((16, DH), jnp.float32),
                       pltpu.SemaphoreType.DMA],
    )
    def k(src_h, dst_h, h_h, ninit_h, out_h, acc, dch, sch, rows_v, sem):
        c = lax.axis_index("c")
        s = lax.axis_index("s")
        w = c * SC_S + s
        lo = w * RW
        pltpu.sync_copy(ninit_h, acc)

        def chunk(ci, carry):
            pltpu.sync_copy(dst_h.at[pl.ds(ci * CH, CH)], dch)
            pltpu.sync_copy(src_h.at[pl.ds(ci * CH, CH)], sch)

            def vreg(i, carry2):
                dv = dch[pl.ds(i * 16, 16)]
                dloc = dv - lo
                msk = (dloc >= 0) & (dloc < RW)
                mi = jnp.where(msk, 1, 0)
                hits = mi[0]
                for lane in range(1, 16):
                    hits = hits + mi[lane]

                @pl.when(hits > 0)
                def _():
                    sv = sch[pl.ds(i * 16, 16)]
                    idxv = jnp.where(msk, sv, 0)
                    dlv = jnp.where(msk, dloc, RW)
                    pltpu.async_copy(h_h.at[idxv], rows_v, sem).wait()
                    for lane in range(16):
                        d = dlv[lane]

                        @pl.when(d < RW)
                        def _(d=d, lane=lane):
                            for j in range(DH // 16):
                                sl = pl.ds(j * 16, 16)
                                acc[d, sl] = jnp.maximum(acc[d, sl],
                                                         rows_v[lane, sl])

                return carry2

            lax.fori_loop(0, VPC, vreg, 0)
            return carry

        lax.fori_loop(0, NCH, chunk, 0)
        pltpu.sync_copy(acc.at[pl.ds(0, RW)], out_h.at[pl.ds(lo, RW)])

    return k(src, dst, h, ninit)


def _sc_gather_rows(tab, idx):
    RW = NP // NW
    C = 80

    @functools.partial(
        pl.kernel,
        out_type=jax.ShapeDtypeStruct((NP, DH), jnp.float32),
        mesh=_sc_mesh(),
        scratch_types=[pltpu.VMEM((C,), jnp.int32),
                       pltpu.VMEM((C, DH), jnp.float32),
                       pltpu.SemaphoreType.DMA],
    )
    def k(tab_h, idx_h, out_h, idx_v, rows_v, sem):
        c = lax.axis_index("c")
        s = lax.axis_index("s")
        base = (c * SC_S + s) * RW

        def body(b, carry):
            off = base + b * C
            pltpu.sync_copy(idx_h.at[pl.ds(off, C)], idx_v)
            pltpu.async_copy(tab_h.at[idx_v], rows_v, sem).wait()
            pltpu.sync_copy(rows_v, out_h.at[pl.ds(off, C)])
            return carry

        lax.fori_loop(0, RW // C, body, 0)

    return k(tab, idx)


def kernel(x, edge_index, timesteps, params):
    p = params
    src = edge_index[0].astype(jnp.int32)
    dst = edge_index[1].astype(jnp.int32)
    ts = timesteps.astype(jnp.int32)

    sort_idx = jnp.argsort(ts).astype(jnp.int32)
    seg_sorted = ts[sort_idx]
    counts = jnp.bincount(ts, length=T).astype(jnp.int32)
    bounds = jnp.concatenate([jnp.zeros((1,), jnp.int32), jnp.cumsum(counts),
                              jnp.array([NP], jnp.int32)]).astype(jnp.int32)
    seg_p = jnp.concatenate([seg_sorted, jnp.full((NP - N,), T, jnp.int32)])
    qi = jnp.arange(NB, dtype=jnp.int32) * BQ
    seg_first = seg_p[qi]
    seg_last = seg_p[qi + BQ - 1]
    kstart = bounds[seg_first]
    kend = bounds[seg_last + 1]
    klo = (kstart // BQ).astype(jnp.int32)
    knum = ((kend - 1) // BQ - klo + 1).astype(jnp.int32)
    zpad = jnp.zeros((NP - N,), jnp.int32)
    sidx_p = jnp.concatenate([sort_idx, zpad])
    pos_p = jnp.concatenate([jnp.argsort(sort_idx).astype(jnp.int32), zpad])
    segc = seg_p.reshape(NP, 1)
    segr = seg_p.reshape(NB, 1, BQ)
    tcol = jnp.concatenate([ts, zpad]).reshape(NP, 1)

    zrow = jnp.zeros((NP, 128), jnp.float32)
    ones128 = jnp.ones((80, 128), jnp.float32)
    ninit = jnp.full((NP // NW + 8, DH), -jnp.inf, jnp.float32)
    temb_pad = jnp.pad(p['temb'], ((0, 64 - T), (0, 0)))

    xp = jnp.pad(x, ((0, NP - N), (0, 0)))
    h = _dense(xp, p['W_in'], p['b_in'], act="relu")

    cnt = None
    for i in (1, 2):
        if cnt is None:
            suma, sumb, cnt = _sc_segsum(src, dst, h[:, :128], h[:, 128:],
                                         zrow, ones128, True)
        else:
            suma, sumb = _sc_segsum(src, dst, h[:, :128], h[:, 128:],
                                    zrow, ones128, False)
        z, st = _sage_mean(suma, sumb, cnt, h, p['sage%d_Wl' % i],
                           p['sage%d_Wr' % i], p['sage%d_bl' % i])
        h = _bn_relu(z, st, p['bn%d_g' % i], p['bn%d_b' % i])

    aggm = _sc_segmax(src, dst, h, ninit)
    z, st = _sage_max(aggm, h, p['sage3_Wl'], p['sage3_Wr'], p['sage3_bl'])
    h = _bn_relu_temb(z, st, p['bn3_g'], p['bn3_b'], tcol, temb_pad)

    hs = _sc_gather_rows(h, sidx_p)
    for l in p['layers']:
        wqkv = jnp.concatenate([l['Wq'], l['Wk'], l['Wv']], axis=1)
        bqkv = jnp.concatenate([l['bq'], l['bk'], l['bv']])
        qkv = _dense(hs, wqkv, bqkv)
        att = _attention(qkv, segc, segr, klo, knum)
        hs = _post(att, hs, l)

    hout = _sc_gather_rows(hs, pos_p)
    y = _cls(hout, p['Wc1'], p['bc1'],
             jnp.pad(p['Wc2'], ((0, 0), (0, 126))), jnp.pad(p['bc2'], (0, 126)))
    return y[:N, :2]


# speedup vs baseline: 40.0994x; 40.0994x over previous
import functools

import jax
import jax.numpy as jnp
from jax import lax
from jax.experimental import pallas as pl
from jax.experimental.pallas import tpu as pltpu
from jax.experimental.pallas import tpu_sc as plsc

N = 10000
NP = 10240
E = 320000
D_IN = 128
DH = 256
T = 50
H = 4
HD = 64
FF = 512
BQ = 256
NB = NP // BQ
NEG = -1e30

SC_C = 2
SC_S = 16
NW = SC_C * SC_S


def _sc_mesh():
    return plsc.VectorSubcoreMesh(core_axis_name="c", subcore_axis_name="s",
                                  num_cores=SC_C, num_subcores=SC_S)


def _mm_kernel(x_ref, w_ref, b_ref, o_ref, *, act):
    z = jnp.dot(x_ref[...], w_ref[...], preferred_element_type=jnp.float32) + b_ref[...]
    if act == "relu":
        z = jnp.maximum(z, 0.0)
    o_ref[...] = z


def _dense(x, w, b, act=None):
    n, k = x.shape
    m = w.shape[1]
    return pl.pallas_call(
        functools.partial(_mm_kernel, act=act),
        grid=(n // BQ,),
        in_specs=[pl.BlockSpec((BQ, k), lambda i: (i, 0)),
                  pl.BlockSpec((k, m), lambda i: (0, 0)),
                  pl.BlockSpec((1, m), lambda i: (0, 0))],
        out_specs=pl.BlockSpec((BQ, m), lambda i: (i, 0)),
        out_shape=jax.ShapeDtypeStruct((n, m), jnp.float32),
    )(x, w, b.reshape(1, m))


def _stats_block(i, z):
    rows = i * BQ + lax.broadcasted_iota(jnp.int32, (BQ, 1), 0)
    zm = jnp.where(rows < N, z, 0.0)
    return jnp.concatenate(
        [jnp.sum(zm, axis=0, keepdims=True),
         jnp.sum(zm * zm, axis=0, keepdims=True),
         jnp.zeros((6, DH), jnp.float32)], axis=0)


def _sage_mean_kernel(suma_ref, sumb_ref, cnt_ref, h_ref, wl_ref, wr_ref, bl_ref,
                      z_ref, st_ref):
    i = pl.program_id(0)
    sa = suma_ref[0] + suma_ref[1]
    sb = sumb_ref[0] + sumb_ref[1]
    c = cnt_ref[0] + cnt_ref[1]
    cc = jnp.maximum(c[:, 0:1], 1.0)
    agg = jnp.concatenate([sa, sb], axis=1) / cc
    z = (jnp.dot(agg, wl_ref[...], preferred_element_type=jnp.float32)
         + jnp.dot(h_ref[...], wr_ref[...], preferred_element_type=jnp.float32)
         + bl_ref[...])
    z_ref[...] = z
    st = _stats_block(i, z)

    @pl.when(i == 0)
    def _():
        st_ref[...] = st

    @pl.when(i > 0)
    def _():
        st_ref[...] = st_ref[...] + st


def _sage_mean(suma, sumb, cnt, h, wl, wr, bl):
    return pl.pallas_call(
        _sage_mean_kernel,
        grid=(NB,),
        in_specs=[pl.BlockSpec((2, BQ, 128), lambda i: (0, i, 0)),
                  pl.BlockSpec((2, BQ, 128), lambda i: (0, i, 0)),
                  pl.BlockSpec((2, BQ, 128), lambda i: (0, i, 0)),
                  pl.BlockSpec((BQ, DH), lambda i: (i, 0)),
                  pl.BlockSpec((DH, DH), lambda i: (0, 0)),
                  pl.BlockSpec((DH, DH), lambda i: (0, 0)),
                  pl.BlockSpec((1, DH), lambda i: (0, 0))],
        out_specs=[pl.BlockSpec((BQ, DH), lambda i: (i, 0)),
                   pl.BlockSpec((8, DH), lambda i: (0, 0))],
        out_shape=[jax.ShapeDtypeStruct((NP, DH), jnp.float32),
                   jax.ShapeDtypeStruct((8, DH), jnp.float32)],
    )(suma, sumb, cnt, h, wl, wr, bl.reshape(1, DH))


def _sage_max_kernel(aggm_ref, h_ref, wl_ref, wr_ref, bl_ref, z_ref, st_ref):
    i = pl.program_id(0)
    a = aggm_ref[...]
    agg = jnp.where(jnp.isfinite(a), a, 0.0)
    z = (jnp.dot(agg, wl_ref[...], preferred_element_type=jnp.float32)
         + jnp.dot(h_ref[...], wr_ref[...], preferred_element_type=jnp.float32)
         + bl_ref[...])
    z_ref[...] = z
    st = _stats_block(i, z)

    @pl.when(i == 0)
    def _():
        st_ref[...] = st

    @pl.when(i > 0)
    def _():
        st_ref[...] = st_ref[...] + st


def _sage_max(aggm, h, wl, wr, bl):
    return pl.pallas_call(
        _sage_max_kernel,
        grid=(NB,),
        in_specs=[pl.BlockSpec((BQ, DH), lambda i: (i, 0)),
                  pl.BlockSpec((BQ, DH), lambda i: (i, 0)),
                  pl.BlockSpec((DH, DH), lambda i: (0, 0)),
                  pl.BlockSpec((DH, DH), lambda i: (0, 0)),
                  pl.BlockSpec((1, DH), lambda i: (0, 0))],
        out_specs=[pl.BlockSpec((BQ, DH), lambda i: (i, 0)),
                   pl.BlockSpec((8, DH), lambda i: (0, 0))],
        out_shape=[jax.ShapeDtypeStruct((NP, DH), jnp.float32),
                   jax.ShapeDtypeStruct((8, DH), jnp.float32)],
    )(aggm, h, wl, wr, bl.reshape(1, DH))


def _bn_kernel(z_ref, st_ref, g_ref, b_ref, o_ref):
    m = st_ref[0:1, :] * (1.0 / N)
    v = st_ref[1:2, :] * (1.0 / N) - m * m
    o_ref[...] = jnp.maximum(
        g_ref[...] * (z_ref[...] - m) * lax.rsqrt(v + 1e-5) + b_ref[...], 0.0)


def _bn_relu(z, st, g, b):
    return pl.pallas_call(
        _bn_kernel,
        grid=(NB,),
        in_specs=[pl.BlockSpec((BQ, DH), lambda i: (i, 0)),
                  pl.BlockSpec((8, DH), lambda i: (0, 0)),
                  pl.BlockSpec((1, DH), lambda i: (0, 0)),
                  pl.BlockSpec((1, DH), lambda i: (0, 0))],
        out_specs=pl.BlockSpec((BQ, DH), lambda i: (i, 0)),
        out_shape=jax.ShapeDtypeStruct((NP, DH), jnp.float32),
    )(z, st, g.reshape(1, DH), b.reshape(1, DH))


def _bn_temb_kernel(z_ref, st_ref, g_ref, b_ref, t_ref, temb_ref, o_ref):
    m = st_ref[0:1, :] * (1.0 / N)
    v = st_ref[1:2, :] * (1.0 / N) - m * m
    bn = jnp.maximum(
        g_ref[...] * (z_ref[...] - m) * lax.rsqrt(v + 1e-5) + b_ref[...], 0.0)
    oh = (t_ref[...] == lax.broadcasted_iota(jnp.int32, (BQ, 64), 1)).astype(jnp.float32)
    o_ref[...] = bn + jnp.dot(oh, temb_ref[...], preferred_element_type=jnp.float32)


def _bn_relu_temb(z, st, g, b, tcol, temb_pad):
    return pl.pallas_call(
        _bn_temb_kernel,
        grid=(NB,),
        in_specs=[pl.BlockSpec((BQ, DH), lambda i: (i, 0)),
                  pl.BlockSpec((8, DH), lambda i: (0, 0)),
                  pl.BlockSpec((1, DH), lambda i: (0, 0)),
                  pl.BlockSpec((1, DH), lambda i: (0, 0)),
                  pl.BlockSpec((BQ, 1), lambda i: (i, 0)),
                  pl.BlockSpec((64, DH), lambda i: (0, 0))],
        out_specs=pl.BlockSpec((BQ, DH), lambda i: (i, 0)),
        out_shape=jax.ShapeDtypeStruct((NP, DH), jnp.float32),
    )(z, st, g.reshape(1, DH), b.reshape(1, DH), tcol, temb_pad)


def _attn_kernel(klo_ref, knum_ref, q_ref, k_ref, v_ref, segc_ref, segr_ref, o_ref):
    i = pl.program_id(0)
    klo = klo_ref[i]
    knum = knum_ref[i]
    segq = segc_ref[...]
    q = q_ref[...] * jnp.float32(0.125)
    for h in range(H):
        qh = q[:, h * HD:(h + 1) * HD]

        def body(j, carry, _h=h, _qh=qh):
            m, l, acc = carry
            kb = klo + j
            krows = k_ref[pl.ds(kb * BQ, BQ), _h * HD:(_h + 1) * HD]
            s = lax.dot_general(_qh, krows, (((1,), (1,)), ((), ())),
                                preferred_element_type=jnp.float32)
            segk = segr_ref[kb]
            s = jnp.where(segq == segk, s, NEG)
            mnew = jnp.maximum(m, jnp.max(s, axis=1, keepdims=True))
            p = jnp.exp(s - mnew)
            corr = jnp.exp(m - mnew)
            vrows = v_ref[pl.ds(kb * BQ, BQ), _h * HD:(_h + 1) * HD]
            l2 = l * corr + jnp.sum(p, axis=1, keepdims=True)
            acc2 = acc * corr + jnp.dot(p, vrows, preferred_element_type=jnp.float32)
            return mnew, l2, acc2

        m0 = jnp.full((BQ, 1), NEG, jnp.float32)
        l0 = jnp.zeros((BQ, 1), jnp.float32)
        a0 = jnp.zeros((BQ, HD), jnp.float32)
        m, l, acc = lax.fori_loop(0, knum, body, (m0, l0, a0))
        o_ref[:, h * HD:(h + 1) * HD] = acc / l


def _attention(qkv, segc, segr, klo, knum):
    return pl.pallas_call(
        _attn_kernel,
        grid=(NB,),
        in_specs=[pl.BlockSpec(memory_space=pltpu.SMEM),
                  pl.BlockSpec(memory_space=pltpu.SMEM),
                  pl.BlockSpec((BQ, DH), lambda i: (i, 0)),
                  pl.BlockSpec((NP, DH), lambda i: (0, 1)),
                  pl.BlockSpec((NP, DH), lambda i: (0, 2)),
                  pl.BlockSpec((BQ, 1), lambda i: (i, 0)),
                  pl.BlockSpec((NB, 1, BQ), lambda i: (0, 0, 0))],
        out_specs=pl.BlockSpec((BQ, DH), lambda i: (i, 0)),
        out_shape=jax.ShapeDtypeStruct((NP, DH), jnp.float32),
    )(klo, knum, qkv, qkv, qkv, segc, segr)


def _post_kernel(a_ref, x_ref, wo_ref, bo_ref, g1_ref, b1_ref, w1_ref, bb1_ref,
                 w2_ref, bb2_ref, g2_ref, b2_ref, o_ref):
    o = (jnp.dot(a_ref[...], wo_ref[...], preferred_element_type=jnp.float32)
         + bo_ref[...] + x_ref[...])
    mu = jnp.mean(o, axis=1, keepdims=True)
    var = jnp.mean((o - mu) * (o - mu), axis=1, keepdims=True)
    u = g1_ref[...] * (o - mu) * lax.rsqrt(var + 1e-5) + b1_ref[...]
    f = jnp.maximum(
        jnp.dot(u, w1_ref[...], preferred_element_type=jnp.float32) + bb1_ref[...], 0.0)
    f = jnp.dot(f, w2_ref[...], preferred_element_type=jnp.float32) + bb2_ref[...] + u
    mu2 = jnp.mean(f, axis=1, keepdims=True)
    var2 = jnp.mean((f - mu2) * (f - mu2), axis=1, keepdims=True)
    o_ref[...] = g2_ref[...] * (f - mu2) * lax.rsqrt(var2 + 1e-5) + b2_ref[...]


def _post(att, hs, l):
    row = lambda a: a.reshape(1, -1)
    return pl.pallas_call(
        _post_kernel,
        grid=(NB,),
        in_specs=[pl.BlockSpec((BQ, DH), lambda i: (i, 0)),
                  pl.BlockSpec((BQ, DH), lambda i: (i, 0)),
                  pl.BlockSpec((DH, DH), lambda i: (0, 0)),
                  pl.BlockSpec((1, DH), lambda i: (0, 0)),
                  pl.BlockSpec((1, DH), lambda i: (0, 0)),
                  pl.BlockSpec((1, DH), lambda i: (0, 0)),
                  pl.BlockSpec((DH, FF), lambda i: (0, 0)),
                  pl.BlockSpec((1, FF), lambda i: (0, 0)),
                  pl.BlockSpec((FF, DH), lambda i: (0, 0)),
                  pl.BlockSpec((1, DH), lambda i: (0, 0)),
                  pl.BlockSpec((1, DH), lambda i: (0, 0)),
                  pl.BlockSpec((1, DH), lambda i: (0, 0))],
        out_specs=pl.BlockSpec((BQ, DH), lambda i: (i, 0)),
        out_shape=jax.ShapeDtypeStruct((NP, DH), jnp.float32),
    )(att, hs, l['Wo'], row(l['bo']), row(l['ln1_g']), row(l['ln1_b']),
      l['W1'], row(l['b1']), l['W2'], row(l['b2']), row(l['ln2_g']), row(l['ln2_b']))


def _cls_kernel(x_ref, w1_ref, b1_ref, w2_ref, b2_ref, o_ref):
    hh = jnp.maximum(
        jnp.dot(x_ref[...], w1_ref[...], preferred_element_type=jnp.float32)
        + b1_ref[...], 0.0)
    o_ref[...] = jnp.dot(hh, w2_ref[...], preferred_element_type=jnp.float32) + b2_ref[...]


def _cls(x, w1, b1, w2p, b2p):
    return pl.pallas_call(
        _cls_kernel,
        grid=(NB,),
        in_specs=[pl.BlockSpec((BQ, DH), lambda i: (i, 0)),
                  pl.BlockSpec((DH, 64), lambda i: (0, 0)),
                  pl.BlockSpec((1, 64), lambda i: (0, 0)),
                  pl.BlockSpec((64, 128), lambda i: (0, 0)),
                  pl.BlockSpec((1, 128), lambda i: (0, 0))],
        out_specs=pl.BlockSpec((BQ, 128), lambda i: (i, 0)),
        out_shape=jax.ShapeDtypeStruct((NP, 128), jnp.float32),
    )(x, w1, b1.reshape(1, 64), w2p, b2p.reshape(1, 128))


def _sc_segsum(src, dst, ha, hb, zrow, ones128, with_cnt):
    TE = E // NW
    C = 80
    NCH = TE // C
    RZ = NP // SC_S
    outs = [jax.ShapeDtypeStruct((SC_C, NP, 128), jnp.float32),
            jax.ShapeDtypeStruct((SC_C, NP, 128), jnp.float32)]
    if with_cnt:
        outs.append(jax.ShapeDtypeStruct((SC_C, NP, 128), jnp.float32))

    @functools.partial(
        pl.kernel,
        out_type=outs,
        mesh=_sc_mesh(),
        scratch_types=[pltpu.VMEM_SHARED((NP, 128), jnp.float32),
                       pltpu.VMEM((C,), jnp.int32),
                       pltpu.VMEM((C,), jnp.int32),
                       pltpu.VMEM((C, 128), jnp.float32),
                       pltpu.VMEM((C, 128), jnp.float32),
                       pltpu.SemaphoreType.DMA],
    )
    def k(src_h, dst_h, ha_h, hb_h, zrow_h, ones_h, *rest):
        if with_cnt:
            suma_h, sumb_h, cnt_h = rest[:3]
            scr = rest[3:]
        else:
            suma_h, sumb_h = rest[:2]
            scr = rest[2:]
        acc_sp, si_v, di_v, rows_v, ones_v, sem = scr
        c = lax.axis_index("c")
        s = lax.axis_index("s")
        ebase = (c * SC_S + s) * TE
        rz = s * RZ

        def zero_acc():
            pltpu.sync_copy(zrow_h.at[pl.ds(rz, RZ)], acc_sp.at[pl.ds(rz, RZ)])

        def sum_pass(h_h, out_h):
            def body(i, carry):
                off = ebase + i * C
                pltpu.sync_copy(src_h.at[pl.ds(off, C)], si_v)
                pltpu.sync_copy(dst_h.at[pl.ds(off, C)], di_v)
                pltpu.async_copy(h_h.at[si_v], rows_v, sem).wait()
                pltpu.sync_copy(rows_v, acc_sp.at[di_v], add=True)
                return carry

            lax.fori_loop(0, NCH, body, 0)
            plsc.subcore_barrier()
            pltpu.sync_copy(acc_sp.at[pl.ds(rz, RZ)], out_h.at[c, pl.ds(rz, RZ)])
            plsc.subcore_barrier()

        zero_acc()
        plsc.subcore_barrier()
        sum_pass(ha_h, suma_h)
        zero_acc()
        plsc.subcore_barrier()
        sum_pass(hb_h, sumb_h)
        if with_cnt:
            pltpu.sync_copy(ones_h, ones_v)
            zero_acc()
            plsc.subcore_barrier()

            def body_c(i, carry):
                off = ebase + i * C
                pltpu.sync_copy(dst_h.at[pl.ds(off, C)], di_v)
                pltpu.sync_copy(ones_v, acc_sp.at[di_v], add=True)
                return carry

            lax.fori_loop(0, NCH, body_c, 0)
            plsc.subcore_barrier()
            pltpu.sync_copy(acc_sp.at[pl.ds(rz, RZ)], cnt_h.at[c, pl.ds(rz, RZ)])

    return k(src, dst, ha, hb, zrow, ones128)


def _sc_segmax(src, dst, h, ninit):
    RW = NP // NW
    AR = RW + 8
    CH = 4000
    NCH = E // CH
    G = 96

    VPC = CH // 16
    NSL = DH // 16

    @functools.partial(
        pl.kernel,
        out_type=jax.ShapeDtypeStruct((NP, DH), jnp.float32),
        mesh=_sc_mesh(),
        scratch_types=[pltpu.VMEM((AR, DH), jnp.float32),
                       pltpu.VMEM((CH,), jnp.int32),
                       pltpu.VMEM((CH,), jnp.int32),
                       pltpu.VMEM((16, DH), jnp.float32),
                       pltpu.SemaphoreType.DMA],
    )
    def k(src_h, dst_h, h_h, ninit_h, out_h, acc, dch, sch, rows_v, sem):
        c = lax.axis_index("c")
        s = lax.axis_index("s")
        w = c * SC_S + s
        lo = w * RW
        hi = lo + RW
        pltpu.sync_copy(ninit_h, acc)

        def chunk(ci, carry):
            pltpu.sync_copy(dst_h.at[pl.ds(ci * CH, CH)], dch)
            first = dch[pl.ds(0, 16)][0]
            last = dch[pl.ds(CH - 16, 16)][15]

            @pl.when((first < hi) & (last >= lo))
            def _():
                pltpu.sync_copy(src_h.at[pl.ds(ci * CH, CH)], sch)

                def vreg(i, carry2):
                    dv = dch[pl.ds(i * 16, 16)]

                    @pl.when((dv[0] < hi) & (dv[15] >= lo))
                    def _():
                        sv = sch[pl.ds(i * 16, 16)]
                        dloc = dv - lo
                        msk = (dloc >= 0) & (dloc < RW)
                        idxv = jnp.where(msk, sv, 0)
                        dlv = jnp.where(msk, dloc, RW)
                        pltpu.async_copy(h_h.at[idxv], rows_v, sem).wait()
                        d = [dlv[t] for t in range(16)]
                        cur = [rows_v[0, pl.ds(j * 16, 16)] for j in range(NSL)]
                        for t in range(1, 16):
                            same = d[t] == d[t - 1]
                            row = [rows_v[t, pl.ds(j * 16, 16)]
                                   for j in range(NSL)]

                            @pl.when(jnp.logical_not(same))
                            def _(dd=d[t - 1], cc=cur):
                                for j in range(NSL):
                                    sl = pl.ds(j * 16, 16)
                                    acc[dd, sl] = jnp.maximum(acc[dd, sl], cc[j])

                            cur = [jnp.where(same, jnp.maximum(cur[j], row[j]),
                                             row[j]) for j in range(NSL)]
                        dd = d[15]
                        for j in range(NSL):
                            sl = pl.ds(j * 16, 16)
                            acc[dd, sl] = jnp.maximum(acc[dd, sl], cur[j])

                    return carry2

                lax.fori_loop(0, VPC, vreg, 0)

            return carry

        lax.fori_loop(0, NCH, chunk, 0)
        pltpu.sync_copy(acc.at[pl.ds(0, RW)], out_h.at[pl.ds(lo, RW)])

    return k(src, dst, h, ninit)


def _sc_gather_rows(tab, idx):
    RW = NP // NW
    C = 80

    @functools.partial(
        pl.kernel,
        out_type=jax.ShapeDtypeStruct((NP, DH), jnp.float32),
        mesh=_sc_mesh(),
        scratch_types=[pltpu.VMEM((C,), jnp.int32),
                       pltpu.VMEM((C, DH), jnp.float32),
                       pltpu.SemaphoreType.DMA],
    )
    def k(tab_h, idx_h, out_h, idx_v, rows_v, sem):
        c = lax.axis_index("c")
        s = lax.axis_index("s")
        base = (c * SC_S + s) * RW

        def body(b, carry):
            off = base + b * C
            pltpu.sync_copy(idx_h.at[pl.ds(off, C)], idx_v)
            pltpu.async_copy(tab_h.at[idx_v], rows_v, sem).wait()
            pltpu.sync_copy(rows_v, out_h.at[pl.ds(off, C)])
            return carry

        lax.fori_loop(0, RW // C, body, 0)

    return k(tab, idx)


def kernel(x, edge_index, timesteps, params):
    p = params
    src = edge_index[0].astype(jnp.int32)
    dst = edge_index[1].astype(jnp.int32)
    ts = timesteps.astype(jnp.int32)

    sort_idx = jnp.argsort(ts).astype(jnp.int32)
    seg_sorted = ts[sort_idx]
    counts = jnp.bincount(ts, length=T).astype(jnp.int32)
    bounds = jnp.concatenate([jnp.zeros((1,), jnp.int32), jnp.cumsum(counts),
                              jnp.array([NP], jnp.int32)]).astype(jnp.int32)
    seg_p = jnp.concatenate([seg_sorted, jnp.full((NP - N,), T, jnp.int32)])
    qi = jnp.arange(NB, dtype=jnp.int32) * BQ
    seg_first = seg_p[qi]
    seg_last = seg_p[qi + BQ - 1]
    kstart = bounds[seg_first]
    kend = bounds[seg_last + 1]
    klo = (kstart // BQ).astype(jnp.int32)
    knum = ((kend - 1) // BQ - klo + 1).astype(jnp.int32)
    zpad = jnp.zeros((NP - N,), jnp.int32)
    sidx_p = jnp.concatenate([sort_idx, zpad])
    pos_p = jnp.concatenate([jnp.argsort(sort_idx).astype(jnp.int32), zpad])
    segc = seg_p.reshape(NP, 1)
    segr = seg_p.reshape(NB, 1, BQ)
    tcol = jnp.concatenate([ts, zpad]).reshape(NP, 1)

    zrow = jnp.zeros((NP, 128), jnp.float32)
    ones128 = jnp.ones((80, 128), jnp.float32)
    ninit = jnp.full((NP // NW + 8, DH), -jnp.inf, jnp.float32)
    temb_pad = jnp.pad(p['temb'], ((0, 64 - T), (0, 0)))

    xp = jnp.pad(x, ((0, NP - N), (0, 0)))
    h = _dense(xp, p['W_in'], p['b_in'], act="relu")

    cnt = None
    for i in (1, 2):
        if cnt is None:
            suma, sumb, cnt = _sc_segsum(src, dst, h[:, :128], h[:, 128:],
                                         zrow, ones128, True)
        else:
            suma, sumb = _sc_segsum(src, dst, h[:, :128], h[:, 128:],
                                    zrow, ones128, False)
        z, st = _sage_mean(suma, sumb, cnt, h, p['sage%d_Wl' % i],
                           p['sage%d_Wr' % i], p['sage%d_bl' % i])
        h = _bn_relu(z, st, p['bn%d_g' % i], p['bn%d_b' % i])

    eorder = jnp.argsort(dst)
    aggm = _sc_segmax(src[eorder], dst[eorder], h, ninit)
    z, st = _sage_max(aggm, h, p['sage3_Wl'], p['sage3_Wr'], p['sage3_bl'])
    h = _bn_relu_temb(z, st, p['bn3_g'], p['bn3_b'], tcol, temb_pad)

    hs = _sc_gather_rows(h, sidx_p)
    for l in p['layers']:
        wqkv = jnp.concatenate([l['Wq'], l['Wk'], l['Wv']], axis=1)
        bqkv = jnp.concatenate([l['bq'], l['bk'], l['bv']])
        qkv = _dense(hs, wqkv, bqkv)
        att = _attention(qkv, segc, segr, klo, knum)
        hs = _post(att, hs, l)

    hout = _sc_gather_rows(hs, pos_p)
    y = _cls(hout, p['Wc1'], p['bc1'],
             jnp.pad(p['Wc2'], ((0, 0), (0, 126))), jnp.pad(p['bc2'], (0, 126)))
    return y[:N, :2]


# speedup vs baseline: 44.4345x; 1.1081x over previous
import functools

import jax
import jax.numpy as jnp
from jax import lax
from jax.experimental import pallas as pl
from jax.experimental.pallas import tpu as pltpu
from jax.experimental.pallas import tpu_sc as plsc

N = 10000
NP = 10240
E = 320000
D_IN = 128
DH = 256
T = 50
H = 4
HD = 64
FF = 512
BQ = 256
NB = NP // BQ
NEG = -1e30

SC_C = 2
SC_S = 16
NW = SC_C * SC_S


def _sc_mesh():
    return plsc.VectorSubcoreMesh(core_axis_name="c", subcore_axis_name="s",
                                  num_cores=SC_C, num_subcores=SC_S)


def _mm_kernel(x_ref, w_ref, b_ref, o_ref, *, act):
    z = jnp.dot(x_ref[...], w_ref[...], preferred_element_type=jnp.float32) + b_ref[...]
    if act == "relu":
        z = jnp.maximum(z, 0.0)
    o_ref[...] = z


def _dense(x, w, b, act=None):
    n, k = x.shape
    m = w.shape[1]
    return pl.pallas_call(
        functools.partial(_mm_kernel, act=act),
        grid=(n // BQ,),
        in_specs=[pl.BlockSpec((BQ, k), lambda i: (i, 0)),
                  pl.BlockSpec((k, m), lambda i: (0, 0)),
                  pl.BlockSpec((1, m), lambda i: (0, 0))],
        out_specs=pl.BlockSpec((BQ, m), lambda i: (i, 0)),
        out_shape=jax.ShapeDtypeStruct((n, m), jnp.float32),
    )(x, w, b.reshape(1, m))


def _stats_block(i, z):
    rows = i * BQ + lax.broadcasted_iota(jnp.int32, (BQ, 1), 0)
    zm = jnp.where(rows < N, z, 0.0)
    return jnp.concatenate(
        [jnp.sum(zm, axis=0, keepdims=True),
         jnp.sum(zm * zm, axis=0, keepdims=True),
         jnp.zeros((6, DH), jnp.float32)], axis=0)


def _sage_mean_kernel(suma_ref, sumb_ref, cnt_ref, h_ref, wl_ref, wr_ref, bl_ref,
                      z_ref, st_ref):
    i = pl.program_id(0)
    sa = suma_ref[0] + suma_ref[1]
    sb = sumb_ref[0] + sumb_ref[1]
    c = cnt_ref[0] + cnt_ref[1]
    cc = jnp.maximum(c[:, 0:1], 1.0)
    agg = jnp.concatenate([sa, sb], axis=1) / cc
    z = (jnp.dot(agg, wl_ref[...], preferred_element_type=jnp.float32)
         + jnp.dot(h_ref[...], wr_ref[...], preferred_element_type=jnp.float32)
         + bl_ref[...])
    z_ref[...] = z
    st = _stats_block(i, z)

    @pl.when(i == 0)
    def _():
        st_ref[...] = st

    @pl.when(i > 0)
    def _():
        st_ref[...] = st_ref[...] + st


def _sage_mean(suma, sumb, cnt, h, wl, wr, bl):
    return pl.pallas_call(
        _sage_mean_kernel,
        grid=(NB,),
        in_specs=[pl.BlockSpec((2, BQ, 128), lambda i: (0, i, 0)),
                  pl.BlockSpec((2, BQ, 128), lambda i: (0, i, 0)),
                  pl.BlockSpec((2, BQ, 128), lambda i: (0, i, 0)),
                  pl.BlockSpec((BQ, DH), lambda i: (i, 0)),
                  pl.BlockSpec((DH, DH), lambda i: (0, 0)),
                  pl.BlockSpec((DH, DH), lambda i: (0, 0)),
                  pl.BlockSpec((1, DH), lambda i: (0, 0))],
        out_specs=[pl.BlockSpec((BQ, DH), lambda i: (i, 0)),
                   pl.BlockSpec((8, DH), lambda i: (0, 0))],
        out_shape=[jax.ShapeDtypeStruct((NP, DH), jnp.float32),
                   jax.ShapeDtypeStruct((8, DH), jnp.float32)],
    )(suma, sumb, cnt, h, wl, wr, bl.reshape(1, DH))


def _sage_max_kernel(aggm_ref, h_ref, wl_ref, wr_ref, bl_ref, z_ref, st_ref):
    i = pl.program_id(0)
    a = aggm_ref[...]
    agg = jnp.where(jnp.isfinite(a), a, 0.0)
    z = (jnp.dot(agg, wl_ref[...], preferred_element_type=jnp.float32)
         + jnp.dot(h_ref[...], wr_ref[...], preferred_element_type=jnp.float32)
         + bl_ref[...])
    z_ref[...] = z
    st = _stats_block(i, z)

    @pl.when(i == 0)
    def _():
        st_ref[...] = st

    @pl.when(i > 0)
    def _():
        st_ref[...] = st_ref[...] + st


def _sage_max(aggm, h, wl, wr, bl):
    return pl.pallas_call(
        _sage_max_kernel,
        grid=(NB,),
        in_specs=[pl.BlockSpec((BQ, DH), lambda i: (i, 0)),
                  pl.BlockSpec((BQ, DH), lambda i: (i, 0)),
                  pl.BlockSpec((DH, DH), lambda i: (0, 0)),
                  pl.BlockSpec((DH, DH), lambda i: (0, 0)),
                  pl.BlockSpec((1, DH), lambda i: (0, 0))],
        out_specs=[pl.BlockSpec((BQ, DH), lambda i: (i, 0)),
                   pl.BlockSpec((8, DH), lambda i: (0, 0))],
        out_shape=[jax.ShapeDtypeStruct((NP, DH), jnp.float32),
                   jax.ShapeDtypeStruct((8, DH), jnp.float32)],
    )(aggm, h, wl, wr, bl.reshape(1, DH))


def _bn_kernel(z_ref, st_ref, g_ref, b_ref, o_ref):
    m = st_ref[0:1, :] * (1.0 / N)
    v = st_ref[1:2, :] * (1.0 / N) - m * m
    o_ref[...] = jnp.maximum(
        g_ref[...] * (z_ref[...] - m) * lax.rsqrt(v + 1e-5) + b_ref[...], 0.0)


def _bn_relu(z, st, g, b):
    return pl.pallas_call(
        _bn_kernel,
        grid=(NB,),
        in_specs=[pl.BlockSpec((BQ, DH), lambda i: (i, 0)),
                  pl.BlockSpec((8, DH), lambda i: (0, 0)),
                  pl.BlockSpec((1, DH), lambda i: (0, 0)),
                  pl.BlockSpec((1, DH), lambda i: (0, 0))],
        out_specs=pl.BlockSpec((BQ, DH), lambda i: (i, 0)),
        out_shape=jax.ShapeDtypeStruct((NP, DH), jnp.float32),
    )(z, st, g.reshape(1, DH), b.reshape(1, DH))


def _bn_temb_kernel(z_ref, st_ref, g_ref, b_ref, t_ref, temb_ref, o_ref):
    m = st_ref[0:1, :] * (1.0 / N)
    v = st_ref[1:2, :] * (1.0 / N) - m * m
    bn = jnp.maximum(
        g_ref[...] * (z_ref[...] - m) * lax.rsqrt(v + 1e-5) + b_ref[...], 0.0)
    oh = (t_ref[...] == lax.broadcasted_iota(jnp.int32, (BQ, 64), 1)).astype(jnp.float32)
    o_ref[...] = bn + jnp.dot(oh, temb_ref[...], preferred_element_type=jnp.float32)


def _bn_relu_temb(z, st, g, b, tcol, temb_pad):
    return pl.pallas_call(
        _bn_temb_kernel,
        grid=(NB,),
        in_specs=[pl.BlockSpec((BQ, DH), lambda i: (i, 0)),
                  pl.BlockSpec((8, DH), lambda i: (0, 0)),
                  pl.BlockSpec((1, DH), lambda i: (0, 0)),
                  pl.BlockSpec((1, DH), lambda i: (0, 0)),
                  pl.BlockSpec((BQ, 1), lambda i: (i, 0)),
                  pl.BlockSpec((64, DH), lambda i: (0, 0))],
        out_specs=pl.BlockSpec((BQ, DH), lambda i: (i, 0)),
        out_shape=jax.ShapeDtypeStruct((NP, DH), jnp.float32),
    )(z, st, g.reshape(1, DH), b.reshape(1, DH), tcol, temb_pad)


def _attn_kernel(klo_ref, knum_ref, q_ref, k_ref, v_ref, segc_ref, segr_ref, o_ref):
    i = pl.program_id(0)
    klo = klo_ref[i]
    knum = knum_ref[i]
    segq = segc_ref[...]
    q = q_ref[...] * jnp.float32(0.125)
    for h in range(H):
        qh = q[:, h * HD:(h + 1) * HD]

        def body(j, carry, _h=h, _qh=qh):
            m, l, acc = carry
            kb = klo + j
            krows = k_ref[pl.ds(kb * BQ, BQ), _h * HD:(_h + 1) * HD]
            s = lax.dot_general(_qh, krows, (((1,), (1,)), ((), ())),
                                preferred_element_type=jnp.float32)
            segk = segr_ref[kb]
            s = jnp.where(segq == segk, s, NEG)
            mnew = jnp.maximum(m, jnp.max(s, axis=1, keepdims=True))
            p = jnp.exp(s - mnew)
            corr = jnp.exp(m - mnew)
            vrows = v_ref[pl.ds(kb * BQ, BQ), _h * HD:(_h + 1) * HD]
            l2 = l * corr + jnp.sum(p, axis=1, keepdims=True)
            acc2 = acc * corr + jnp.dot(p, vrows, preferred_element_type=jnp.float32)
            return mnew, l2, acc2

        m0 = jnp.full((BQ, 1), NEG, jnp.float32)
        l0 = jnp.zeros((BQ, 1), jnp.float32)
        a0 = jnp.zeros((BQ, HD), jnp.float32)
        m, l, acc = lax.fori_loop(0, knum, body, (m0, l0, a0))
        o_ref[:, h * HD:(h + 1) * HD] = acc / l


def _attention(qkv, segc, segr, klo, knum):
    return pl.pallas_call(
        _attn_kernel,
        grid=(NB,),
        in_specs=[pl.BlockSpec(memory_space=pltpu.SMEM),
                  pl.BlockSpec(memory_space=pltpu.SMEM),
                  pl.BlockSpec((BQ, DH), lambda i: (i, 0)),
                  pl.BlockSpec((NP, DH), lambda i: (0, 1)),
                  pl.BlockSpec((NP, DH), lambda i: (0, 2)),
                  pl.BlockSpec((BQ, 1), lambda i: (i, 0)),
                  pl.BlockSpec((NB, 1, BQ), lambda i: (0, 0, 0))],
        out_specs=pl.BlockSpec((BQ, DH), lambda i: (i, 0)),
        out_shape=jax.ShapeDtypeStruct((NP, DH), jnp.float32),
    )(klo, knum, qkv, qkv, qkv, segc, segr)


def _post_kernel(a_ref, x_ref, wo_ref, bo_ref, g1_ref, b1_ref, w1_ref, bb1_ref,
                 w2_ref, bb2_ref, g2_ref, b2_ref, o_ref):
    o = (jnp.dot(a_ref[...], wo_ref[...], preferred_element_type=jnp.float32)
         + bo_ref[...] + x_ref[...])
    mu = jnp.mean(o, axis=1, keepdims=True)
    var = jnp.mean((o - mu) * (o - mu), axis=1, keepdims=True)
    u = g1_ref[...] * (o - mu) * lax.rsqrt(var + 1e-5) + b1_ref[...]
    f = jnp.maximum(
        jnp.dot(u, w1_ref[...], preferred_element_type=jnp.float32) + bb1_ref[...], 0.0)
    f = jnp.dot(f, w2_ref[...], preferred_element_type=jnp.float32) + bb2_ref[...] + u
    mu2 = jnp.mean(f, axis=1, keepdims=True)
    var2 = jnp.mean((f - mu2) * (f - mu2), axis=1, keepdims=True)
    o_ref[...] = g2_ref[...] * (f - mu2) * lax.rsqrt(var2 + 1e-5) + b2_ref[...]


def _post(att, hs, l):
    row = lambda a: a.reshape(1, -1)
    return pl.pallas_call(
        _post_kernel,
        grid=(NB,),
        in_specs=[pl.BlockSpec((BQ, DH), lambda i: (i, 0)),
                  pl.BlockSpec((BQ, DH), lambda i: (i, 0)),
                  pl.BlockSpec((DH, DH), lambda i: (0, 0)),
                  pl.BlockSpec((1, DH), lambda i: (0, 0)),
                  pl.BlockSpec((1, DH), lambda i: (0, 0)),
                  pl.BlockSpec((1, DH), lambda i: (0, 0)),
                  pl.BlockSpec((DH, FF), lambda i: (0, 0)),
                  pl.BlockSpec((1, FF), lambda i: (0, 0)),
                  pl.BlockSpec((FF, DH), lambda i: (0, 0)),
                  pl.BlockSpec((1, DH), lambda i: (0, 0)),
                  pl.BlockSpec((1, DH), lambda i: (0, 0)),
                  pl.BlockSpec((1, DH), lambda i: (0, 0))],
        out_specs=pl.BlockSpec((BQ, DH), lambda i: (i, 0)),
        out_shape=jax.ShapeDtypeStruct((NP, DH), jnp.float32),
    )(att, hs, l['Wo'], row(l['bo']), row(l['ln1_g']), row(l['ln1_b']),
      l['W1'], row(l['b1']), l['W2'], row(l['b2']), row(l['ln2_g']), row(l['ln2_b']))


def _cls_kernel(x_ref, w1_ref, b1_ref, w2_ref, b2_ref, o_ref):
    hh = jnp.maximum(
        jnp.dot(x_ref[...], w1_ref[...], preferred_element_type=jnp.float32)
        + b1_ref[...], 0.0)
    o_ref[...] = jnp.dot(hh, w2_ref[...], preferred_element_type=jnp.float32) + b2_ref[...]


def _cls(x, w1, b1, w2p, b2p):
    return pl.pallas_call(
        _cls_kernel,
        grid=(NB,),
        in_specs=[pl.BlockSpec((BQ, DH), lambda i: (i, 0)),
                  pl.BlockSpec((DH, 64), lambda i: (0, 0)),
                  pl.BlockSpec((1, 64), lambda i: (0, 0)),
                  pl.BlockSpec((64, 128), lambda i: (0, 0)),
                  pl.BlockSpec((1, 128), lambda i: (0, 0))],
        out_specs=pl.BlockSpec((BQ, 128), lambda i: (i, 0)),
        out_shape=jax.ShapeDtypeStruct((NP, 128), jnp.float32),
    )(x, w1, b1.reshape(1, 64), w2p, b2p.reshape(1, 128))


def _sc_segsum(src, dst, ha, hb, zrow, ones128, with_cnt):
    TE = E // NW
    C = 80
    NCH = TE // C
    RZ = NP // SC_S
    outs = [jax.ShapeDtypeStruct((SC_C, NP, 128), jnp.float32),
            jax.ShapeDtypeStruct((SC_C, NP, 128), jnp.float32)]
    if with_cnt:
        outs.append(jax.ShapeDtypeStruct((SC_C, NP, 128), jnp.float32))

    @functools.partial(
        pl.kernel,
        out_type=outs,
        mesh=_sc_mesh(),
        scratch_types=[pltpu.VMEM_SHARED((NP, 128), jnp.float32),
                       pltpu.VMEM((C,), jnp.int32),
                       pltpu.VMEM((C,), jnp.int32),
                       pltpu.VMEM((C,), jnp.int32),
                       pltpu.VMEM((C,), jnp.int32),
                       pltpu.VMEM((C, 128), jnp.float32),
                       pltpu.VMEM((C, 128), jnp.float32),
                       pltpu.VMEM((C, 128), jnp.float32),
                       pltpu.SemaphoreType.DMA,
                       pltpu.SemaphoreType.DMA],
    )
    def k(src_h, dst_h, ha_h, hb_h, zrow_h, ones_h, *rest):
        if with_cnt:
            suma_h, sumb_h, cnt_h = rest[:3]
            scr = rest[3:]
        else:
            suma_h, sumb_h = rest[:2]
            scr = rest[2:]
        acc_sp, si_v0, si_v1, di_v0, di_v1, rows_v0, rows_v1, ones_v, sem0, sem1 = scr
        si_v = (si_v0, si_v1)
        di_v = (di_v0, di_v1)
        rows_v = (rows_v0, rows_v1)
        sems = (sem0, sem1)
        c = lax.axis_index("c")
        s = lax.axis_index("s")
        ebase = (c * SC_S + s) * TE
        rz = s * RZ

        def zero_acc():
            pltpu.sync_copy(zrow_h.at[pl.ds(rz, RZ)], acc_sp.at[pl.ds(rz, RZ)])

        def sum_pass(h_h, out_h):
            def pair(p, carry):
                for b in (0, 1):
                    i = 2 * p + b

                    @pl.when(i < NCH)
                    def _(i=i, b=b):
                        off = ebase + i * C
                        pltpu.sync_copy(src_h.at[pl.ds(off, C)], si_v[b])
                        pltpu.sync_copy(dst_h.at[pl.ds(off, C)], di_v[b])
                        pltpu.make_async_copy(h_h.at[si_v[b]], rows_v[b],
                                              sems[b]).start()

                for b in (0, 1):
                    i = 2 * p + b

                    @pl.when(i < NCH)
                    def _(i=i, b=b):
                        pltpu.make_async_copy(h_h.at[si_v[b]], rows_v[b],
                                              sems[b]).wait()
                        pltpu.sync_copy(rows_v[b], acc_sp.at[di_v[b]], add=True)

                return carry

            lax.fori_loop(0, (NCH + 1) // 2, pair, 0)
            plsc.subcore_barrier()
            pltpu.sync_copy(acc_sp.at[pl.ds(rz, RZ)], out_h.at[c, pl.ds(rz, RZ)])
            plsc.subcore_barrier()

        zero_acc()
        plsc.subcore_barrier()
        sum_pass(ha_h, suma_h)
        zero_acc()
        plsc.subcore_barrier()
        sum_pass(hb_h, sumb_h)
        if with_cnt:
            pltpu.sync_copy(ones_h, ones_v)
            zero_acc()
            plsc.subcore_barrier()

            def body_c(i, carry):
                off = ebase + i * C
                pltpu.sync_copy(dst_h.at[pl.ds(off, C)], di_v[0])
                pltpu.sync_copy(ones_v, acc_sp.at[di_v[0]], add=True)
                return carry

            lax.fori_loop(0, NCH, body_c, 0)
            plsc.subcore_barrier()
            pltpu.sync_copy(acc_sp.at[pl.ds(rz, RZ)], cnt_h.at[c, pl.ds(rz, RZ)])

    return k(src, dst, ha, hb, zrow, ones128)


def _sc_segmax(src, dst, h, ninit):
    RW = NP // NW
    AR = RW + 8
    CH = 4000
    NCH = E // CH
    G = 96

    VPC = CH // 16
    NSL = DH // 16

    @functools.partial(
        pl.kernel,
        out_type=jax.ShapeDtypeStruct((NP, DH), jnp.float32),
        mesh=_sc_mesh(),
        scratch_types=[pltpu.VMEM((AR, DH), jnp.float32),
                       pltpu.VMEM((CH,), jnp.int32),
                       pltpu.VMEM((CH,), jnp.int32),
                       pltpu.VMEM((16, DH), jnp.float32),
                       pltpu.SemaphoreType.DMA],
    )
    def k(src_h, dst_h, h_h, ninit_h, out_h, acc, dch, sch, rows_v, sem):
        c = lax.axis_index("c")
        s = lax.axis_index("s")
        w = c * SC_S + s
        lo = w * RW
        hi = lo + RW
        pltpu.sync_copy(ninit_h, acc)

        def chunk(ci, carry):
            pltpu.sync_copy(dst_h.at[pl.ds(ci * CH, CH)], dch)
            first = dch[pl.ds(0, 16)][0]
            last = dch[pl.ds(CH - 16, 16)][15]

            @pl.when((first < hi) & (last >= lo))
            def _():
                pltpu.sync_copy(src_h.at[pl.ds(ci * CH, CH)], sch)

                def vreg(i, carry2):
                    dv = dch[pl.ds(i * 16, 16)]

                    @pl.when((dv[0] < hi) & (dv[15] >= lo))
                    def _():
                        sv = sch[pl.ds(i * 16, 16)]
                        dloc = dv - lo
                        msk = (dloc >= 0) & (dloc < RW)
                        idxv = jnp.where(msk, sv, 0)
                        dlv = jnp.where(msk, dloc, RW)
                        pltpu.async_copy(h_h.at[idxv], rows_v, sem).wait()
                        d = [dlv[t] for t in range(16)]
                        cur = [rows_v[0, pl.ds(j * 16, 16)] for j in range(NSL)]
                        for t in range(1, 16):
                            same = d[t] == d[t - 1]
                            row = [rows_v[t, pl.ds(j * 16, 16)]
                                   for j in range(NSL)]

                            @pl.when(jnp.logical_not(same))
                            def _(dd=d[t - 1], cc=cur):
                                for j in range(NSL):
                                    sl = pl.ds(j * 16, 16)
                                    acc[dd, sl] = jnp.maximum(acc[dd, sl], cc[j])

                            cur = [jnp.where(same, jnp.maximum(cur[j], row[j]),
                                             row[j]) for j in range(NSL)]
                        dd = d[15]
                        for j in range(NSL):
                            sl = pl.ds(j * 16, 16)
                            acc[dd, sl] = jnp.maximum(acc[dd, sl], cur[j])

                    return carry2

                lax.fori_loop(0, VPC, vreg, 0)

            return carry

        lax.fori_loop(0, NCH, chunk, 0)
        pltpu.sync_copy(acc.at[pl.ds(0, RW)], out_h.at[pl.ds(lo, RW)])

    return k(src, dst, h, ninit)


def _sc_gather_rows(tab, idx):
    RW = NP // NW
    C = 80

    @functools.partial(
        pl.kernel,
        out_type=jax.ShapeDtypeStruct((NP, DH), jnp.float32),
        mesh=_sc_mesh(),
        scratch_types=[pltpu.VMEM((C,), jnp.int32),
                       pltpu.VMEM((C, DH), jnp.float32),
                       pltpu.SemaphoreType.DMA],
    )
    def k(tab_h, idx_h, out_h, idx_v, rows_v, sem):
        c = lax.axis_index("c")
        s = lax.axis_index("s")
        base = (c * SC_S + s) * RW

        def body(b, carry):
            off = base + b * C
            pltpu.sync_copy(idx_h.at[pl.ds(off, C)], idx_v)
            pltpu.async_copy(tab_h.at[idx_v], rows_v, sem).wait()
            pltpu.sync_copy(rows_v, out_h.at[pl.ds(off, C)])
            return carry

        lax.fori_loop(0, RW // C, body, 0)

    return k(tab, idx)


def kernel(x, edge_index, timesteps, params):
    p = params
    src = edge_index[0].astype(jnp.int32)
    dst = edge_index[1].astype(jnp.int32)
    ts = timesteps.astype(jnp.int32)

    sort_idx = jnp.argsort(ts).astype(jnp.int32)
    seg_sorted = ts[sort_idx]
    counts = jnp.bincount(ts, length=T).astype(jnp.int32)
    bounds = jnp.concatenate([jnp.zeros((1,), jnp.int32), jnp.cumsum(counts),
                              jnp.array([NP], jnp.int32)]).astype(jnp.int32)
    seg_p = jnp.concatenate([seg_sorted, jnp.full((NP - N,), T, jnp.int32)])
    qi = jnp.arange(NB, dtype=jnp.int32) * BQ
    seg_first = seg_p[qi]
    seg_last = seg_p[qi + BQ - 1]
    kstart = bounds[seg_first]
    kend = bounds[seg_last + 1]
    klo = (kstart // BQ).astype(jnp.int32)
    knum = ((kend - 1) // BQ - klo + 1).astype(jnp.int32)
    zpad = jnp.zeros((NP - N,), jnp.int32)
    sidx_p = jnp.concatenate([sort_idx, zpad])
    pos_p = jnp.concatenate([jnp.argsort(sort_idx).astype(jnp.int32), zpad])
    segc = seg_p.reshape(NP, 1)
    segr = seg_p.reshape(NB, 1, BQ)
    tcol = jnp.concatenate([ts, zpad]).reshape(NP, 1)

    zrow = jnp.zeros((NP, 128), jnp.float32)
    ones128 = jnp.ones((80, 128), jnp.float32)
    ninit = jnp.full((NP // NW + 8, DH), -jnp.inf, jnp.float32)
    temb_pad = jnp.pad(p['temb'], ((0, 64 - T), (0, 0)))

    xp = jnp.pad(x, ((0, NP - N), (0, 0)))
    h = _dense(xp, p['W_in'], p['b_in'], act="relu")

    cnt = None
    for i in (1, 2):
        if cnt is None:
            suma, sumb, cnt = _sc_segsum(src, dst, h[:, :128], h[:, 128:],
                                         zrow, ones128, True)
        else:
            suma, sumb = _sc_segsum(src, dst, h[:, :128], h[:, 128:],
                                    zrow, ones128, False)
        z, st = _sage_mean(suma, sumb, cnt, h, p['sage%d_Wl' % i],
                           p['sage%d_Wr' % i], p['sage%d_bl' % i])
        h = _bn_relu(z, st, p['bn%d_g' % i], p['bn%d_b' % i])

    eorder = jnp.argsort(dst)
    aggm = _sc_segmax(src[eorder], dst[eorder], h, ninit)
    z, st = _sage_max(aggm, h, p['sage3_Wl'], p['sage3_Wr'], p['sage3_bl'])
    h = _bn_relu_temb(z, st, p['bn3_g'], p['bn3_b'], tcol, temb_pad)

    hs = _sc_gather_rows(h, sidx_p)
    for l in p['layers']:
        wqkv = jnp.concatenate([l['Wq'], l['Wk'], l['Wv']], axis=1)
        bqkv = jnp.concatenate([l['bq'], l['bk'], l['bv']])
        qkv = _dense(hs, wqkv, bqkv)
        att = _attention(qkv, segc, segr, klo, knum)
        hs = _post(att, hs, l)

    hout = _sc_gather_rows(hs, pos_p)
    y = _cls(hout, p['Wc1'], p['bc1'],
             jnp.pad(p['Wc2'], ((0, 0), (0, 126))), jnp.pad(p['bc2'], (0, 126)))
    return y[:N, :2]
